# initial kernel scaffold (unmeasured)
import jax
import jax.numpy as jnp
from jax import lax
from jax.experimental import pallas as pl
from jax.experimental.pallas import tpu as pltpu


def kernel(
    x,
):
    def body(*refs):
        pass

    out_shape = jax.ShapeDtypeStruct(..., jnp.float32)
    return pl.pallas_call(body, out_shape=out_shape)(...)


# baseline (device time: 55897 ns/iter reference)
import jax
import jax.numpy as jnp
from jax import lax
from jax.experimental import pallas as pl
from jax.experimental.pallas import tpu as pltpu

M = 1024
N = 1024


def kernel(x):
    def body(x_ref, out_ref, acc_ref, ybuf_ref, xbuf_ref, send_sems, recv_sems):
        my_x = lax.axis_index("x")
        my_y = lax.axis_index("y")
        y_nbr = (my_x, 1 - my_y)
        x_nbr = (1 - my_x, my_y)

        barrier = pltpu.get_barrier_semaphore()
        for nbr in (y_nbr, x_nbr):
            pl.semaphore_signal(
                barrier, inc=1, device_id=nbr,
                device_id_type=pl.DeviceIdType.MESH,
            )
        pl.semaphore_wait(barrier, 2)

        acc_ref[...] = x_ref[0, 0].astype(jnp.bfloat16)

        rdma1 = pltpu.make_async_remote_copy(
            src_ref=acc_ref,
            dst_ref=ybuf_ref,
            send_sem=send_sems.at[0],
            recv_sem=recv_sems.at[0],
            device_id=y_nbr,
            device_id_type=pl.DeviceIdType.MESH,
        )
        rdma1.start()
        rdma1.wait()
        acc_ref[...] = acc_ref[...] + ybuf_ref[...]

        rdma2 = pltpu.make_async_remote_copy(
            src_ref=acc_ref,
            dst_ref=xbuf_ref,
            send_sem=send_sems.at[1],
            recv_sem=recv_sems.at[1],
            device_id=x_nbr,
            device_id_type=pl.DeviceIdType.MESH,
        )
        rdma2.start()
        rdma2.wait()
        out_ref[...] = (acc_ref[...] + xbuf_ref[...]).astype(jnp.float32)

    return pl.pallas_call(
        body,
        out_shape=jax.ShapeDtypeStruct((M, N), jnp.float32),
        in_specs=[pl.BlockSpec(memory_space=pltpu.VMEM)],
        out_specs=pl.BlockSpec(memory_space=pltpu.VMEM),
        scratch_shapes=[
            pltpu.VMEM((M, N), jnp.bfloat16),
            pltpu.VMEM((M, N), jnp.bfloat16),
            pltpu.VMEM((M, N), jnp.bfloat16),
            pltpu.SemaphoreType.DMA((2,)),
            pltpu.SemaphoreType.DMA((2,)),
        ],
        compiler_params=pltpu.CompilerParams(collective_id=0),
    )(x)


# device time: 31210 ns/iter; 1.7910x vs baseline; 1.7910x over previous
import jax
import jax.numpy as jnp
from jax import lax
from jax.experimental import pallas as pl
from jax.experimental.pallas import tpu as pltpu

M = 1024
N = 1024
H = 512
Q = 256
E = 128


def kernel(x):
    def body(x_ref, out_ref, acc_ref, rsa1, rsb1, rsa2, rsb2,
             send_sems, recv_sems):
        my_x = lax.axis_index("x")
        my_y = lax.axis_index("y")
        y_nbr = (my_x, 1 - my_y)
        x_nbr = (1 - my_x, my_y)

        barrier = pltpu.get_barrier_semaphore()
        for nbr in (y_nbr, x_nbr):
            pl.semaphore_signal(
                barrier, inc=1, device_id=nbr,
                device_id_type=pl.DeviceIdType.MESH,
            )
        pl.semaphore_wait(barrier, 2)

        acc_ref[...] = x_ref[0, 0].astype(jnp.bfloat16)

        a1 = my_y * Q
        b1 = H + my_x * Q
        aown = a1 + my_x * E
        bown = b1 + my_y * E

        def exchange(src, dst, sem_idx, nbr):
            return pltpu.make_async_remote_copy(
                src_ref=src, dst_ref=dst,
                send_sem=send_sems.at[sem_idx],
                recv_sem=recv_sems.at[sem_idx],
                device_id=nbr, device_id_type=pl.DeviceIdType.MESH,
            )

        a = exchange(acc_ref.at[pl.ds((1 - my_y) * Q, Q)], rsa1, 0, y_nbr)
        b = exchange(acc_ref.at[pl.ds(H + (1 - my_x) * Q, Q)], rsb1, 4, x_nbr)
        a.start()
        b.start()
        a.wait()
        acc_ref[pl.ds(a1, Q), :] = acc_ref[pl.ds(a1, Q), :] + rsa1[...]
        b.wait()
        acc_ref[pl.ds(b1, Q), :] = acc_ref[pl.ds(b1, Q), :] + rsb1[...]

        a = exchange(acc_ref.at[pl.ds(a1 + (1 - my_x) * E, E)], rsa2, 1, x_nbr)
        b = exchange(acc_ref.at[pl.ds(b1 + (1 - my_y) * E, E)], rsb2, 5, y_nbr)
        a.start()
        b.start()
        a.wait()
        acc_ref[pl.ds(aown, E), :] = acc_ref[pl.ds(aown, E), :] + rsa2[...]
        b.wait()
        acc_ref[pl.ds(bown, E), :] = acc_ref[pl.ds(bown, E), :] + rsb2[...]

        a = exchange(acc_ref.at[pl.ds(aown, E)], acc_ref.at[pl.ds(aown, E)],
                     2, x_nbr)
        b = exchange(acc_ref.at[pl.ds(bown, E)], acc_ref.at[pl.ds(bown, E)],
                     6, y_nbr)
        a.start()
        b.start()
        a.wait()
        b.wait()

        a = exchange(acc_ref.at[pl.ds(a1, Q)], acc_ref.at[pl.ds(a1, Q)],
                     3, y_nbr)
        b = exchange(acc_ref.at[pl.ds(b1, Q)], acc_ref.at[pl.ds(b1, Q)],
                     7, x_nbr)
        a.start()
        b.start()
        a.wait()
        b.wait()

        out_ref[...] = acc_ref[...].astype(jnp.float32)

    return pl.pallas_call(
        body,
        out_shape=jax.ShapeDtypeStruct((M, N), jnp.float32),
        in_specs=[pl.BlockSpec(memory_space=pltpu.VMEM)],
        out_specs=pl.BlockSpec(memory_space=pltpu.VMEM),
        scratch_shapes=[
            pltpu.VMEM((M, N), jnp.bfloat16),
            pltpu.VMEM((Q, N), jnp.bfloat16),
            pltpu.VMEM((Q, N), jnp.bfloat16),
            pltpu.VMEM((E, N), jnp.bfloat16),
            pltpu.VMEM((E, N), jnp.bfloat16),
            pltpu.SemaphoreType.DMA((8,)),
            pltpu.SemaphoreType.DMA((8,)),
        ],
        compiler_params=pltpu.CompilerParams(collective_id=0),
    )(x)


# device time: 28841 ns/iter; 1.9381x vs baseline; 1.0821x over previous
import jax
import jax.numpy as jnp
from jax import lax
from jax.experimental import pallas as pl
from jax.experimental.pallas import tpu as pltpu

M = 1024
N = 1024
H = 512
Q = 256
E = 128


def kernel(x):
    def body(x_ref, o_ref, rsa1, rsb1, rsa2, rsb2, send_sems, recv_sems):
        my_x = lax.axis_index("x")
        my_y = lax.axis_index("y")
        y_nbr = (my_x, 1 - my_y)
        x_nbr = (1 - my_x, my_y)

        barrier = pltpu.get_barrier_semaphore()
        for nbr in (y_nbr, x_nbr):
            pl.semaphore_signal(
                barrier, inc=1, device_id=nbr,
                device_id_type=pl.DeviceIdType.MESH,
            )
        pl.semaphore_wait(barrier, 2)

        a1 = my_y * Q
        b1 = H + my_x * Q
        a_send = (1 - my_y) * Q
        b_send = H + (1 - my_x) * Q
        aown = a1 + my_x * E
        aoth = a1 + (1 - my_x) * E
        bown = b1 + my_y * E
        both = b1 + (1 - my_y) * E

        def ex(src, dst, i, nbr):
            return pltpu.make_async_remote_copy(
                src_ref=src, dst_ref=dst,
                send_sem=send_sems.at[i], recv_sem=recv_sems.at[i],
                device_id=nbr, device_id_type=pl.DeviceIdType.MESH,
            )

        o_ref[pl.ds(a_send, Q), :] = x_ref[0, 0, pl.ds(a_send, Q), :].astype(
            jnp.bfloat16)
        a_rs1c0 = ex(o_ref.at[pl.ds(a_send + (1 - my_x) * E, E)],
                     rsa1.at[pl.ds(0, E)], 0, y_nbr)
        a_rs1c1 = ex(o_ref.at[pl.ds(a_send + my_x * E, E)],
                     rsa1.at[pl.ds(E, E)], 1, y_nbr)
        a_rs1c0.start()
        a_rs1c1.start()
        o_ref[pl.ds(b_send, Q), :] = x_ref[0, 0, pl.ds(b_send, Q), :].astype(
            jnp.bfloat16)
        b_rs1c0 = ex(o_ref.at[pl.ds(b_send + (1 - my_y) * E, E)],
                     rsb1.at[pl.ds(0, E)], 6, x_nbr)
        b_rs1c1 = ex(o_ref.at[pl.ds(b_send + my_y * E, E)],
                     rsb1.at[pl.ds(E, E)], 7, x_nbr)
        b_rs1c0.start()
        b_rs1c1.start()
        o_ref[pl.ds(a1, Q), :] = x_ref[0, 0, pl.ds(a1, Q), :].astype(
            jnp.bfloat16)
        o_ref[pl.ds(b1, Q), :] = x_ref[0, 0, pl.ds(b1, Q), :].astype(
            jnp.bfloat16)

        a_rs1c0.wait_recv()
        o_ref[pl.ds(aoth, E), :] = o_ref[pl.ds(aoth, E), :] + rsa1[pl.ds(0, E), :]
        a_rs2 = ex(o_ref.at[pl.ds(aoth, E)], rsa2, 2, x_nbr)
        a_rs2.start()
        b_rs1c0.wait_recv()
        o_ref[pl.ds(both, E), :] = o_ref[pl.ds(both, E), :] + rsb1[pl.ds(0, E), :]
        b_rs2 = ex(o_ref.at[pl.ds(both, E)], rsb2, 8, y_nbr)
        b_rs2.start()

        a_rs1c1.wait_recv()
        o_ref[pl.ds(aown, E), :] = o_ref[pl.ds(aown, E), :] + rsa1[pl.ds(E, E), :]
        b_rs1c1.wait_recv()
        o_ref[pl.ds(bown, E), :] = o_ref[pl.ds(bown, E), :] + rsb1[pl.ds(E, E), :]

        a_rs2.wait_recv()
        o_ref[pl.ds(aown, E), :] = o_ref[pl.ds(aown, E), :] + rsa2[...]
        a_ag1 = ex(o_ref.at[pl.ds(aown, E)], o_ref.at[pl.ds(aown, E)], 3, x_nbr)
        a_ag2a = ex(o_ref.at[pl.ds(aown, E)], o_ref.at[pl.ds(aown, E)], 4, y_nbr)
        a_ag1.start()
        a_ag2a.start()
        b_rs2.wait_recv()
        o_ref[pl.ds(bown, E), :] = o_ref[pl.ds(bown, E), :] + rsb2[...]
        b_ag1 = ex(o_ref.at[pl.ds(bown, E)], o_ref.at[pl.ds(bown, E)], 9, y_nbr)
        b_ag2a = ex(o_ref.at[pl.ds(bown, E)], o_ref.at[pl.ds(bown, E)], 10, x_nbr)
        b_ag1.start()
        b_ag2a.start()

        a_ag1.wait_recv()
        a_ag2b = ex(o_ref.at[pl.ds(aoth, E)], o_ref.at[pl.ds(aoth, E)], 5, y_nbr)
        a_ag2b.start()
        b_ag1.wait_recv()
        b_ag2b = ex(o_ref.at[pl.ds(both, E)], o_ref.at[pl.ds(both, E)], 11, x_nbr)
        b_ag2b.start()

        a_ag2a.wait_recv()
        a_ag2b.wait_recv()
        b_ag2a.wait_recv()
        b_ag2b.wait_recv()

        for d in (a_rs1c0, a_rs1c1, a_rs2, a_ag1, a_ag2a, a_ag2b,
                  b_rs1c0, b_rs1c1, b_rs2, b_ag1, b_ag2a, b_ag2b):
            d.wait_send()

    return pl.pallas_call(
        body,
        out_shape=jax.ShapeDtypeStruct((M, N), jnp.bfloat16),
        in_specs=[pl.BlockSpec(memory_space=pltpu.VMEM)],
        out_specs=pl.BlockSpec(memory_space=pltpu.VMEM),
        scratch_shapes=[
            pltpu.VMEM((Q, N), jnp.bfloat16),
            pltpu.VMEM((Q, N), jnp.bfloat16),
            pltpu.VMEM((E, N), jnp.bfloat16),
            pltpu.VMEM((E, N), jnp.bfloat16),
            pltpu.SemaphoreType.DMA((12,)),
            pltpu.SemaphoreType.DMA((12,)),
        ],
        compiler_params=pltpu.CompilerParams(collective_id=0),
    )(x)
